# baseline (speedup 1.0000x reference)
import jax
import jax.numpy as jnp
from jax import lax
from jax.experimental import pallas as pl
from jax.experimental.pallas import tpu as pltpu
from jax.experimental.pallas import tpu_sc as plsc

_M = 1024
_N = 4
_TOT = _M * _N
_K = 128
_GAMMA = 0.5
_EPS = 0.5
_BSZ = 16
_NT = 16
_CHUNK = _TOT // _NT
_GPT = _M // _NT
_NV = _CHUNK // 16
_BIG = 3.4e38


def _za_body(x_ref, wa_ref, ba_ref, out_ref):
    acc = lax.dot_general(x_ref[...], wa_ref[...],
                          (((1,), (1,)), ((), ())),
                          preferred_element_type=jnp.float32)
    out_ref[...] = acc + ba_ref[...]


def _pred_body(y_ref, wd_ref, bd_ref, out_ref):
    acc = lax.dot_general(y_ref[...], wd_ref[...],
                          (((1,), (1,)), ((), ())),
                          preferred_element_type=jnp.float32)
    out_ref[...] = acc + bd_ref[...]


def _tr_body(wb_ref, out_ref):
    out_ref[:, 0, 0, :] = jnp.transpose(wb_ref[...], (1, 0))


def _build_table(W_b):
    out = pl.pallas_call(
        _tr_body,
        grid=(_NT, 8),
        in_specs=[pl.BlockSpec((_CHUNK, _TOT // 8), lambda t, jc: (t, jc))],
        out_specs=pl.BlockSpec((_TOT // 8, 1, 1, _CHUNK),
                               lambda t, jc: (jc, t, 0, 0)),
        out_shape=jax.ShapeDtypeStruct((_TOT, _NT, 1, _CHUNK), jnp.float32),
        compiler_params=pltpu.CompilerParams(
            dimension_semantics=("arbitrary", "arbitrary")),
    )(W_b)
    return out.reshape(_TOT * _NT, _CHUNK)


def _tanh_via_exp(v):
    return 1.0 - 2.0 / (jnp.exp(2.0 * v) + 1.0)


def _sc_body(za_hbm, wg_hbm, bb_hbm,
             y_hbm, xb_hbm, phi_hbm, psi_hbm,
             za_v, bb_v, u_v, psi_v, phi_v, sig_v,
             sstar_v, lam_v, lami_v, jstar_v, yrow_v, jj_v, dv_v, dphi_v,
             lamall_v, cand_v, jall_v, dvall_v,
             cidx_v, cidxb_v, cdv_v, rowsa_v, rowsb_v, red_v, tmp_v,
             sh_red, sh_lam, sh_j, sh_dv,
             *dma_sems):
    wid = lax.axis_index("s")
    lanes = lax.iota(jnp.int32, 16)
    zeros16 = jnp.zeros((16,), jnp.float32)

    pltpu.sync_copy(bb_hbm.at[wid], bb_v)
    pltpu.sync_copy(za_hbm.at[wid], za_v)
    for k in range(_NV):
        u_v[pl.ds(k * 16, 16)] = zeros16
        psi_v[pl.ds(k * 16, 16)] = zeros16
        phi_v[pl.ds(k * 16, 16)] = zeros16
    for k in range(_K // 16):
        cidx_v[pl.ds(k * 16, 16)] = jnp.zeros((16,), jnp.int32)

    def step(i, s_carry):
        s = s_carry
        alpha = jnp.where(s == 0.0, jnp.float32(1.0), s)
        inv_a = 1.0 / (jnp.full((16,), 1.0) * alpha)

        mnv = jnp.full((16,), _BIG)
        for k in range(_NV):
            zav = za_v[pl.ds(i * _CHUNK + k * 16, 16)]
            sg = (zav + u_v[pl.ds(k * 16, 16)] * inv_a
                  + bb_v[pl.ds(k * 16, 16)])
            sig_v[pl.ds(k * 16, 16)] = sg
            mnv = jnp.minimum(mnv, sg)
        tmp_v[...] = mnv
        pltpu.sync_copy(tmp_v, sh_red.at[wid])
        plsc.subcore_barrier()

        pltpu.sync_copy(sh_red, red_v)
        gm = jnp.full((16,), _BIG)
        for t in range(_NT):
            gm = jnp.minimum(gm, red_v[t])
        gmin = jnp.min(gm)

        for k in range(_GPT // 16):
            base = (k * 16 + lanes) * 4
            best = jnp.full((16,), -_BIG)
            beststar = zeros16
            bestj = jnp.zeros((16,), jnp.int32)
            for n in range(_N):
                sgn = plsc.load_gather(sig_v, [base + n])
                phn = plsc.load_gather(phi_v, [base + n])
                pin = (1.0 - phn) * (sgn - gmin + 1.0)
                upd = pin > best
                best = jnp.where(upd, pin, best)
                beststar = jnp.where(upd, sgn, beststar)
                bestj = jnp.where(upd, base + n, bestj)
            lam_v[pl.ds(k * 16, 16)] = best
            lami_v[pl.ds(k * 16, 16)] = plsc.bitcast(best, jnp.int32)
            sstar_v[pl.ds(k * 16, 16)] = beststar
            jstar_v[pl.ds(k * 16, 16)] = bestj
        pltpu.sync_copy(lami_v, sh_lam.at[pl.ds(wid * _GPT, _GPT)])
        plsc.subcore_barrier()

        pltpu.sync_copy(sh_lam, lamall_v)

        def scan_count(cand):
            def cnt_iter(k, cv):
                return cv + jnp.where(
                    lamall_v[pl.ds(k * 16, 16)] >= cand, 1.0, 0.0)
            return jnp.sum(
                lax.fori_loop(0, _M // 16, cnt_iter, zeros16, unroll=8))

        def bit_hi(bi, t_acc):
            cand = t_acc | (jnp.int32(1) << (30 - bi))
            return jnp.where(scan_count(cand) >= jnp.float32(_K),
                             cand, t_acc)

        t10 = lax.fori_loop(0, 10, bit_hi, jnp.int32(0))
        up = t10 + (jnp.int32(1) << 21)
        hi = scan_count(up)

        for k in range(_M // 16):
            cand_v[pl.ds(k * 16, 16)] = jnp.zeros((16,), jnp.int32)

        def comp2(k, cntf):
            v = lamall_v[pl.ds(k * 16, 16)]
            m = jnp.logical_and(v >= t10, v < up)
            mi = jnp.where(m, 1, 0).astype(jnp.int32)
            pos = cntf.astype(jnp.int32) + plsc.cumsum(mi) - 1
            plsc.store_scatter(cand_v, [pos], v, mask=m)
            return cntf + jnp.sum(jnp.where(m, 1.0, 0.0))

        ncf = lax.fori_loop(0, _M // 16, comp2, jnp.float32(0.0))
        nvreg = (ncf.astype(jnp.int32) + 15) >> 4

        def bit_lo(bi, t_acc):
            cand = t_acc | (jnp.int32(1) << (20 - bi))

            def cnt2(k, cv):
                return cv + jnp.where(
                    cand_v[pl.ds(k * 16, 16)] >= cand, 1.0, 0.0)

            cnt = hi + jnp.sum(lax.fori_loop(0, nvreg, cnt2, zeros16))
            return jnp.where(cnt >= jnp.float32(_K), cand, t_acc)

        thr = lax.fori_loop(0, 21, bit_lo, t10)

        for k in range(_GPT // 16):
            sl = pl.ds(k * 16, 16)
            sel = lami_v[sl] >= thr
            self_f = jnp.where(sel, 1.0, 0.0)
            yv = _tanh_via_exp(sstar_v[sl]) * self_f
            yrow_v[sl] = jnp.maximum(yv, 0.0)
            jst = jstar_v[sl]
            psj = plsc.load_gather(psi_v, [jst])
            phj = plsc.load_gather(phi_v, [jst])
            dv_v[sl] = jnp.maximum(yv - psj * _EPS, 0.0)
            dphi_v[sl] = jnp.maximum(yv - phj * _GAMMA, 0.0)
            jj_v[sl] = jst + wid * _CHUNK
        pltpu.sync_copy(jj_v, sh_j.at[pl.ds(wid * _GPT, _GPT)])
        pltpu.sync_copy(dv_v, sh_dv.at[pl.ds(wid * _GPT, _GPT)])

        for k in range(_NV):
            sl = pl.ds(k * 16, 16)
            psi_v[sl] = psi_v[sl] * _EPS
            phi_v[sl] = phi_v[sl] * _GAMMA
        for k in range(_GPT // 16):
            sl = pl.ds(k * 16, 16)
            jst = jstar_v[sl]
            plsc.addupdate_scatter(psi_v, [jst], dv_v[sl])
            plsc.addupdate_scatter(phi_v, [jst], dphi_v[sl])
        plsc.subcore_barrier()

        pltpu.sync_copy(sh_j, jall_v)
        pltpu.sync_copy(sh_dv, dvall_v)
        for k in range(_K // 16):
            cdv_v[pl.ds(k * 16, 16)] = zeros16

        def comp_iter(k, carry):
            cntf, sumd = carry
            dv = dvall_v[pl.ds(k * 16, 16)]
            jv = jall_v[pl.ds(k * 16, 16)]
            m = dv > 0.0
            mi = jnp.where(m, 1, 0).astype(jnp.int32)
            pos = cntf.astype(jnp.int32) + plsc.cumsum(mi) - 1
            gidx = jv * (2 * _NT) + 2 * wid
            plsc.store_scatter(cidx_v, [pos], gidx, mask=m)
            plsc.store_scatter(cdv_v, [pos], dv, mask=m)
            return (cntf + jnp.sum(jnp.where(m, 1.0, 0.0)),
                    sumd + jnp.sum(dv))

        _, sumd = lax.fori_loop(0, _M // 16, comp_iter,
                                (jnp.float32(0.0), jnp.float32(0.0)))

        for k in range(_K // 16):
            cidxb_v[pl.ds(k * 16, 16)] = (
                cidx_v[pl.ds(k * 16, 16)] + jnp.int32(1))
        nch = _K // 32
        descs = []
        for ch in range(nch):
            descs.append((
                pltpu.async_copy(wg_hbm.at[cidx_v.at[pl.ds(ch * 32, 32)]],
                                 rowsa_v.at[pl.ds(ch * 32, 32)],
                                 dma_sems[2 * ch]),
                pltpu.async_copy(wg_hbm.at[cidxb_v.at[pl.ds(ch * 32, 32)]],
                                 rowsb_v.at[pl.ds(ch * 32, 32)],
                                 dma_sems[2 * ch + 1]),
            ))

        def acc_iter(r, accs):
            wv = plsc.load_gather(cdv_v, [jnp.full((16,), r, jnp.int32)])
            new = [accs[k] + wv * rowsa_v[r, pl.ds(k * 16, 16)]
                   for k in range(_NV // 2)]
            new += [accs[_NV // 2 + k] + wv * rowsb_v[r, pl.ds(k * 16, 16)]
                    for k in range(_NV // 2)]
            return tuple(new)

        accs = tuple(u_v[pl.ds(k * 16, 16)] * _EPS for k in range(_NV // 2))
        accs = accs + tuple(
            u_v[pl.ds((_NV // 2 + k) * 16, 16)] * _EPS
            for k in range(_NV // 2))
        for ch in range(nch):
            descs[ch][0].wait()
            descs[ch][1].wait()
            accs = lax.fori_loop(ch * 32, (ch + 1) * 32, acc_iter, accs,
                                 unroll=2)
        for k in range(_NV):
            u_v[pl.ds(k * 16, 16)] = accs[k]

        pltpu.sync_copy(yrow_v, y_hbm.at[i, wid])
        return _EPS * s + sumd

    s_fin = lax.fori_loop(0, _BSZ, step, jnp.float32(0.0))

    alpha_f = jnp.where(s_fin == 0.0, jnp.float32(1.0), s_fin)
    inv_f = 1.0 / (jnp.full((16,), 1.0) * alpha_f)
    for k in range(_NV):
        sig_v[pl.ds(k * 16, 16)] = psi_v[pl.ds(k * 16, 16)] * inv_f
    pltpu.sync_copy(sig_v, xb_hbm.at[wid])
    pltpu.sync_copy(phi_v, phi_hbm.at[wid])
    pltpu.sync_copy(psi_v, psi_hbm.at[wid])


def _run_sc(za_t, wg, bb_t):
    mesh = plsc.VectorSubcoreMesh(core_axis_name="c", subcore_axis_name="s",
                                  num_cores=1, num_subcores=_NT)
    f = pl.kernel(
        _sc_body,
        out_type=(
            jax.ShapeDtypeStruct((_BSZ, _NT, _GPT), jnp.float32),
            jax.ShapeDtypeStruct((_NT, _CHUNK), jnp.float32),
            jax.ShapeDtypeStruct((_NT, _CHUNK), jnp.float32),
            jax.ShapeDtypeStruct((_NT, _CHUNK), jnp.float32),
        ),
        mesh=mesh,
        compiler_params=pltpu.CompilerParams(
            needs_layout_passes=False, use_tc_tiling_on_sc=False),
        scratch_types=[
            pltpu.VMEM((_BSZ * _CHUNK,), jnp.float32),
            pltpu.VMEM((_CHUNK,), jnp.float32),
            pltpu.VMEM((_CHUNK,), jnp.float32),
            pltpu.VMEM((_CHUNK,), jnp.float32),
            pltpu.VMEM((_CHUNK,), jnp.float32),
            pltpu.VMEM((_CHUNK,), jnp.float32),
            pltpu.VMEM((_GPT,), jnp.float32),
            pltpu.VMEM((_GPT,), jnp.float32),
            pltpu.VMEM((_GPT,), jnp.int32),
            pltpu.VMEM((_GPT,), jnp.int32),
            pltpu.VMEM((_GPT,), jnp.float32),
            pltpu.VMEM((_GPT,), jnp.int32),
            pltpu.VMEM((_GPT,), jnp.float32),
            pltpu.VMEM((_GPT,), jnp.float32),
            pltpu.VMEM((_M,), jnp.int32),
            pltpu.VMEM((_M,), jnp.int32),
            pltpu.VMEM((_M,), jnp.int32),
            pltpu.VMEM((_M,), jnp.float32),
            pltpu.VMEM((_K,), jnp.int32),
            pltpu.VMEM((_K,), jnp.int32),
            pltpu.VMEM((_K,), jnp.float32),
            pltpu.VMEM((_K, _CHUNK // 2), jnp.float32),
            pltpu.VMEM((_K, _CHUNK // 2), jnp.float32),
            pltpu.VMEM((_NT, 16), jnp.float32),
            pltpu.VMEM((16,), jnp.float32),
            pltpu.VMEM_SHARED((_NT, 16), jnp.float32),
            pltpu.VMEM_SHARED((_M,), jnp.int32),
            pltpu.VMEM_SHARED((_M,), jnp.int32),
            pltpu.VMEM_SHARED((_M,), jnp.float32),
            pltpu.SemaphoreType.DMA,
            pltpu.SemaphoreType.DMA,
            pltpu.SemaphoreType.DMA,
            pltpu.SemaphoreType.DMA,
            pltpu.SemaphoreType.DMA,
            pltpu.SemaphoreType.DMA,
            pltpu.SemaphoreType.DMA,
            pltpu.SemaphoreType.DMA,
        ],
    )
    return f(za_t, wg, bb_t)


def kernel(batch_x, W_a, b_a, W_b, b_b, W_d, b_d):
    za = pl.pallas_call(
        _za_body,
        out_shape=jax.ShapeDtypeStruct((_BSZ, _M), jnp.float32),
    )(batch_x, W_a, b_a.reshape(1, _M))

    wg = W_b.T.reshape(_TOT * _NT * 2, _CHUNK // 2)
    za4 = jnp.repeat(za, _N, axis=1)
    za_t = (za4.reshape(_BSZ, _NT, _CHUNK).transpose(1, 0, 2)
            .reshape(_NT, _BSZ * _CHUNK))
    bb_t = b_b.reshape(_NT, _CHUNK)

    y_out, xb_out, phi_out, psi_out = _run_sc(za_t, wg, bb_t)

    preds = pl.pallas_call(
        _pred_body,
        out_shape=jax.ShapeDtypeStruct((_BSZ, 1024), jnp.float32),
    )(y_out.reshape(_BSZ, _M), W_d, b_d.reshape(1, 1024))

    xb = xb_out.reshape(_TOT)
    phi = phi_out.reshape(_M, _N)
    psi = psi_out.reshape(_M, _N)
    return preds, xb, phi, psi


# speedup vs baseline: 1.0647x; 1.0647x over previous
import jax
import jax.numpy as jnp
from jax import lax
from jax.experimental import pallas as pl
from jax.experimental.pallas import tpu as pltpu
from jax.experimental.pallas import tpu_sc as plsc

_M = 1024
_N = 4
_TOT = _M * _N
_K = 128
_GAMMA = 0.5
_EPS = 0.5
_BSZ = 16
_NT = 16
_CHUNK = _TOT // _NT
_GPT = _M // _NT
_NV = _CHUNK // 16
_BIG = 3.4e38


def _za_body(x_ref, wa_ref, ba_ref, out_ref):
    acc = lax.dot_general(x_ref[...], wa_ref[...],
                          (((1,), (1,)), ((), ())),
                          preferred_element_type=jnp.float32)
    out_ref[...] = acc + ba_ref[...]


def _pred_body(y_ref, wd_ref, bd_ref, out_ref):
    acc = lax.dot_general(y_ref[...], wd_ref[...],
                          (((1,), (1,)), ((), ())),
                          preferred_element_type=jnp.float32)
    out_ref[...] = acc + bd_ref[...]


def _tr_body(wb_ref, out_ref):
    out_ref[:, 0, 0, :] = jnp.transpose(wb_ref[...], (1, 0))


def _build_table(W_b):
    out = pl.pallas_call(
        _tr_body,
        grid=(_NT, 8),
        in_specs=[pl.BlockSpec((_CHUNK, _TOT // 8), lambda t, jc: (t, jc))],
        out_specs=pl.BlockSpec((_TOT // 8, 1, 1, _CHUNK),
                               lambda t, jc: (jc, t, 0, 0)),
        out_shape=jax.ShapeDtypeStruct((_TOT, _NT, 1, _CHUNK), jnp.float32),
        compiler_params=pltpu.CompilerParams(
            dimension_semantics=("arbitrary", "arbitrary")),
    )(W_b)
    return out.reshape(_TOT * _NT, _CHUNK)


def _tanh_via_exp(v):
    return 1.0 - 2.0 / (jnp.exp(2.0 * v) + 1.0)


def _sc_body(za_hbm, wg_hbm, bb_hbm,
             y_hbm, xb_hbm, phi_hbm, psi_hbm,
             za_v, bb_v, u_v, psi_v, phi_v, sig_v,
             sstar_v, lam_v, jstar_v, yrow_v, jj_v, dv_v, dphi_v,
             lamall_v, jall_v, dvall_v,
             cidx_v, cidxb_v, cdv_v, rowsa_v, rowsb_v, red_v, tmp_v,
             sh_red, sh_lam, sh_j, sh_dv,
             *dma_sems):
    wid = lax.axis_index("s")
    lanes = lax.iota(jnp.int32, 16)
    zeros16 = jnp.zeros((16,), jnp.float32)

    pltpu.sync_copy(bb_hbm.at[wid], bb_v)
    pltpu.sync_copy(za_hbm.at[wid], za_v)
    for k in range(_NV):
        u_v[pl.ds(k * 16, 16)] = zeros16
        psi_v[pl.ds(k * 16, 16)] = zeros16
        phi_v[pl.ds(k * 16, 16)] = zeros16
    for k in range(_K // 16):
        cidx_v[pl.ds(k * 16, 16)] = jnp.zeros((16,), jnp.int32)

    def step(i, s_carry):
        s = s_carry
        alpha = jnp.where(s == 0.0, jnp.float32(1.0), s)
        inv_a = 1.0 / (jnp.full((16,), 1.0) * alpha)

        mnv = jnp.full((16,), _BIG)
        for k in range(_NV):
            zav = za_v[pl.ds(i * _CHUNK + k * 16, 16)]
            sg = (zav + u_v[pl.ds(k * 16, 16)] * inv_a
                  + bb_v[pl.ds(k * 16, 16)])
            sig_v[pl.ds(k * 16, 16)] = sg
            mnv = jnp.minimum(mnv, sg)
        tmp_v[...] = mnv
        pltpu.sync_copy(tmp_v, sh_red.at[wid])
        plsc.subcore_barrier()

        pltpu.sync_copy(sh_red, red_v)
        gm = jnp.full((16,), _BIG)
        for t in range(_NT):
            gm = jnp.minimum(gm, red_v[t])
        gmin = jnp.min(gm)

        for k in range(_GPT // 16):
            base = (k * 16 + lanes) * 4
            best = jnp.full((16,), -_BIG)
            beststar = zeros16
            bestj = jnp.zeros((16,), jnp.int32)
            for n in range(_N):
                sgn = plsc.load_gather(sig_v, [base + n])
                phn = plsc.load_gather(phi_v, [base + n])
                pin = (1.0 - phn) * (sgn - gmin + 1.0)
                upd = pin > best
                best = jnp.where(upd, pin, best)
                beststar = jnp.where(upd, sgn, beststar)
                bestj = jnp.where(upd, base + n, bestj)
            lam_v[pl.ds(k * 16, 16)] = best
            sstar_v[pl.ds(k * 16, 16)] = beststar
            jstar_v[pl.ds(k * 16, 16)] = bestj
        pltpu.sync_copy(lam_v, sh_lam.at[pl.ds(wid * _GPT, _GPT)])
        plsc.subcore_barrier()

        pltpu.sync_copy(sh_lam, lamall_v)

        def bit_iter(bi, t_acc):
            cand = t_acc | (jnp.int32(1) << (30 - bi))

            def cnt_iter(k, cv):
                b = plsc.bitcast(lamall_v[pl.ds(k * 16, 16)], jnp.int32)
                return cv + jnp.where(b >= cand, 1.0, 0.0)

            cv = lax.fori_loop(0, _M // 16, cnt_iter, zeros16, unroll=8)
            return jnp.where(jnp.sum(cv) >= jnp.float32(_K), cand, t_acc)

        thr = lax.fori_loop(0, 31, bit_iter, jnp.int32(0))

        for k in range(_GPT // 16):
            sl = pl.ds(k * 16, 16)
            sel = plsc.bitcast(lam_v[sl], jnp.int32) >= thr
            self_f = jnp.where(sel, 1.0, 0.0)
            yv = _tanh_via_exp(sstar_v[sl]) * self_f
            yrow_v[sl] = jnp.maximum(yv, 0.0)
            jst = jstar_v[sl]
            psj = plsc.load_gather(psi_v, [jst])
            phj = plsc.load_gather(phi_v, [jst])
            dv_v[sl] = jnp.maximum(yv - psj * _EPS, 0.0)
            dphi_v[sl] = jnp.maximum(yv - phj * _GAMMA, 0.0)
            jj_v[sl] = jst + wid * _CHUNK
        pltpu.sync_copy(jj_v, sh_j.at[pl.ds(wid * _GPT, _GPT)])
        pltpu.sync_copy(dv_v, sh_dv.at[pl.ds(wid * _GPT, _GPT)])

        for k in range(_NV):
            sl = pl.ds(k * 16, 16)
            psi_v[sl] = psi_v[sl] * _EPS
            phi_v[sl] = phi_v[sl] * _GAMMA
        for k in range(_GPT // 16):
            sl = pl.ds(k * 16, 16)
            jst = jstar_v[sl]
            plsc.addupdate_scatter(psi_v, [jst], dv_v[sl])
            plsc.addupdate_scatter(phi_v, [jst], dphi_v[sl])
        plsc.subcore_barrier()

        pltpu.sync_copy(sh_j, jall_v)
        pltpu.sync_copy(sh_dv, dvall_v)
        for k in range(_K // 16):
            cdv_v[pl.ds(k * 16, 16)] = zeros16

        def comp_iter(k, carry):
            cntf, sumd = carry
            dv = dvall_v[pl.ds(k * 16, 16)]
            jv = jall_v[pl.ds(k * 16, 16)]
            m = dv > 0.0
            mi = jnp.where(m, 1, 0).astype(jnp.int32)
            pos = cntf.astype(jnp.int32) + plsc.cumsum(mi) - 1
            gidx = jv * (2 * _NT) + 2 * wid
            plsc.store_scatter(cidx_v, [pos], gidx, mask=m)
            plsc.store_scatter(cdv_v, [pos], dv, mask=m)
            return (cntf + jnp.sum(jnp.where(m, 1.0, 0.0)),
                    sumd + jnp.sum(dv))

        _, sumd = lax.fori_loop(0, _M // 16, comp_iter,
                                (jnp.float32(0.0), jnp.float32(0.0)))

        for k in range(_K // 16):
            cidxb_v[pl.ds(k * 16, 16)] = (
                cidx_v[pl.ds(k * 16, 16)] + jnp.int32(1))
        nch = _K // 32
        descs = []
        for ch in range(nch):
            descs.append((
                pltpu.async_copy(wg_hbm.at[cidx_v.at[pl.ds(ch * 32, 32)]],
                                 rowsa_v.at[pl.ds(ch * 32, 32)],
                                 dma_sems[2 * ch]),
                pltpu.async_copy(wg_hbm.at[cidxb_v.at[pl.ds(ch * 32, 32)]],
                                 rowsb_v.at[pl.ds(ch * 32, 32)],
                                 dma_sems[2 * ch + 1]),
            ))

        def acc_iter(r, accs):
            wv = plsc.load_gather(cdv_v, [jnp.full((16,), r, jnp.int32)])
            new = [accs[k] + wv * rowsa_v[r, pl.ds(k * 16, 16)]
                   for k in range(_NV // 2)]
            new += [accs[_NV // 2 + k] + wv * rowsb_v[r, pl.ds(k * 16, 16)]
                    for k in range(_NV // 2)]
            return tuple(new)

        accs = tuple(u_v[pl.ds(k * 16, 16)] * _EPS for k in range(_NV // 2))
        accs = accs + tuple(
            u_v[pl.ds((_NV // 2 + k) * 16, 16)] * _EPS
            for k in range(_NV // 2))
        for ch in range(nch):
            descs[ch][0].wait()
            descs[ch][1].wait()
            accs = lax.fori_loop(ch * 32, (ch + 1) * 32, acc_iter, accs,
                                 unroll=2)
        for k in range(_NV):
            u_v[pl.ds(k * 16, 16)] = accs[k]

        pltpu.sync_copy(yrow_v, y_hbm.at[i, wid])
        return _EPS * s + sumd

    s_fin = lax.fori_loop(0, _BSZ, step, jnp.float32(0.0))

    alpha_f = jnp.where(s_fin == 0.0, jnp.float32(1.0), s_fin)
    inv_f = 1.0 / (jnp.full((16,), 1.0) * alpha_f)
    for k in range(_NV):
        sig_v[pl.ds(k * 16, 16)] = psi_v[pl.ds(k * 16, 16)] * inv_f
    pltpu.sync_copy(sig_v, xb_hbm.at[wid])
    pltpu.sync_copy(phi_v, phi_hbm.at[wid])
    pltpu.sync_copy(psi_v, psi_hbm.at[wid])


def _run_sc(za_t, wg, bb_t):
    mesh = plsc.VectorSubcoreMesh(core_axis_name="c", subcore_axis_name="s",
                                  num_cores=1, num_subcores=_NT)
    f = pl.kernel(
        _sc_body,
        out_type=(
            jax.ShapeDtypeStruct((_BSZ, _NT, _GPT), jnp.float32),
            jax.ShapeDtypeStruct((_NT, _CHUNK), jnp.float32),
            jax.ShapeDtypeStruct((_NT, _CHUNK), jnp.float32),
            jax.ShapeDtypeStruct((_NT, _CHUNK), jnp.float32),
        ),
        mesh=mesh,
        compiler_params=pltpu.CompilerParams(
            needs_layout_passes=False, use_tc_tiling_on_sc=False),
        scratch_types=[
            pltpu.VMEM((_BSZ * _CHUNK,), jnp.float32),
            pltpu.VMEM((_CHUNK,), jnp.float32),
            pltpu.VMEM((_CHUNK,), jnp.float32),
            pltpu.VMEM((_CHUNK,), jnp.float32),
            pltpu.VMEM((_CHUNK,), jnp.float32),
            pltpu.VMEM((_CHUNK,), jnp.float32),
            pltpu.VMEM((_GPT,), jnp.float32),
            pltpu.VMEM((_GPT,), jnp.float32),
            pltpu.VMEM((_GPT,), jnp.int32),
            pltpu.VMEM((_GPT,), jnp.float32),
            pltpu.VMEM((_GPT,), jnp.int32),
            pltpu.VMEM((_GPT,), jnp.float32),
            pltpu.VMEM((_GPT,), jnp.float32),
            pltpu.VMEM((_M,), jnp.float32),
            pltpu.VMEM((_M,), jnp.int32),
            pltpu.VMEM((_M,), jnp.float32),
            pltpu.VMEM((_K,), jnp.int32),
            pltpu.VMEM((_K,), jnp.int32),
            pltpu.VMEM((_K,), jnp.float32),
            pltpu.VMEM((_K, _CHUNK // 2), jnp.float32),
            pltpu.VMEM((_K, _CHUNK // 2), jnp.float32),
            pltpu.VMEM((_NT, 16), jnp.float32),
            pltpu.VMEM((16,), jnp.float32),
            pltpu.VMEM_SHARED((_NT, 16), jnp.float32),
            pltpu.VMEM_SHARED((_M,), jnp.float32),
            pltpu.VMEM_SHARED((_M,), jnp.int32),
            pltpu.VMEM_SHARED((_M,), jnp.float32),
            pltpu.SemaphoreType.DMA,
            pltpu.SemaphoreType.DMA,
            pltpu.SemaphoreType.DMA,
            pltpu.SemaphoreType.DMA,
            pltpu.SemaphoreType.DMA,
            pltpu.SemaphoreType.DMA,
            pltpu.SemaphoreType.DMA,
            pltpu.SemaphoreType.DMA,
        ],
    )
    return f(za_t, wg, bb_t)


def kernel(batch_x, W_a, b_a, W_b, b_b, W_d, b_d):
    za = pl.pallas_call(
        _za_body,
        out_shape=jax.ShapeDtypeStruct((_BSZ, _M), jnp.float32),
    )(batch_x, W_a, b_a.reshape(1, _M))

    wg = W_b.T.reshape(_TOT * _NT * 2, _CHUNK // 2)
    za4 = jnp.repeat(za, _N, axis=1)
    za_t = (za4.reshape(_BSZ, _NT, _CHUNK).transpose(1, 0, 2)
            .reshape(_NT, _BSZ * _CHUNK))
    bb_t = b_b.reshape(_NT, _CHUNK)

    y_out, xb_out, phi_out, psi_out = _run_sc(za_t, wg, bb_t)

    preds = pl.pallas_call(
        _pred_body,
        out_shape=jax.ShapeDtypeStruct((_BSZ, 1024), jnp.float32),
    )(y_out.reshape(_BSZ, _M), W_d, b_d.reshape(1, 1024))

    xb = xb_out.reshape(_TOT)
    phi = phi_out.reshape(_M, _N)
    psi = psi_out.reshape(_M, _N)
    return preds, xb, phi, psi
